# baseline (speedup 1.0000x reference)
import functools

import jax
import jax.numpy as jnp
from jax.experimental import pallas as pl

N_BITS = 24
NUM_SAMPLES = 262144
_ROWS = 2048


def _pack_body(probs_ref, u_ref, keys_ref):
    u = u_ref[...]
    probs = probs_ref[...]
    bits = (u < probs[None, :]).astype(jnp.int32)
    powers = (1 << jax.lax.broadcasted_iota(jnp.int32, (1, N_BITS), 1))
    keys_ref[...] = jnp.sum(bits * powers, axis=1)


def _pack_keys(probs, u):
    n = u.shape[0]
    return pl.pallas_call(
        _pack_body,
        grid=(n // _ROWS,),
        in_specs=[
            pl.BlockSpec((N_BITS,), lambda i: (0,)),
            pl.BlockSpec((_ROWS, N_BITS), lambda i: (i, 0)),
        ],
        out_specs=pl.BlockSpec((_ROWS,), lambda i: (i,)),
        out_shape=jax.ShapeDtypeStruct((n,), jnp.int32),
    )(probs, u)


def kernel(kernel, u, num_samples):
    probs = jax.nn.sigmoid(2.0 * kernel)
    packed = _pack_keys(probs, u)
    size = u.shape[0]
    uniq, counts = jnp.unique(packed, return_counts=True, size=size, fill_value=-1)
    counts = jnp.minimum(counts, num_samples)
    valid = counts > 0
    bit_idx = jnp.arange(N_BITS, dtype=jnp.int32)
    bits = ((uniq[:, None] >> bit_idx[None, :]) & 1).astype(jnp.int8)
    bits = jnp.where(valid[:, None], bits, jnp.int8(0))
    return bits, counts.astype(jnp.int32)
